# baseline (speedup 1.0000x reference)
import jax
import jax.numpy as jnp
from jax.experimental import pallas as pl
from jax.experimental.pallas import tpu as pltpu

N = 256
E = 1024
F = 128
U = 16
KA = 264


def _body(eip_ref, x_ref, xT_ref,
          w1a0T_ref, w1a128_ref, w1a129_ref, b1a_ref,
          w1b0_ref, w1bL_ref, b1b_ref, b1bL_ref,
          w0a0_ref, w0aL_ref, b0a_ref, w0b_ref, b0b_ref,
          out_ref, B_s, G_s, MH_s):
    f32 = jnp.float32
    bf16 = jnp.bfloat16
    src_row = eip_ref[0:1, :]
    dst_row = eip_ref[1:2, :]
    xT = xT_ref[...]

    node_iota = jax.lax.broadcasted_iota(jnp.int32, (N, E), 0)
    Sd = (node_iota == dst_row).astype(bf16)
    Ss = (node_iota == src_row).astype(bf16)
    nt = (((1,), (1,)), ((), ()))
    cnt = jax.lax.dot_general(Sd, Ss, nt, preferred_element_type=f32)
    cntT = jax.lax.dot_general(Ss, Sd, nt, preferred_element_type=f32)
    r = jax.lax.broadcasted_iota(jnp.int32, (N, N), 0)
    c = jax.lax.broadcasted_iota(jnp.int32, (N, N), 1)
    offdiag = (r != c).astype(f32)
    B = (cnt > 0).astype(f32) * offdiag
    Bt = (cntT > 0).astype(f32) * offdiag
    Bbf = B.astype(bf16)
    G = jax.lax.dot_general(Bbf, Bbf, nt, preferred_element_type=f32)
    B_s[...] = B
    G_s[...] = G
    kdeg = jnp.sum(B, axis=1, keepdims=True)

    w1a128 = w1a128_ref[...]
    wsum = w1a128 + w1a129_ref[...]
    b1a = b1a_ref[...]
    w1b0 = w1b0_ref[...]
    w1bL = w1bL_ref[...]
    b1b = b1b_ref[...]
    b1bL = b1bL_ref[...]
    w0a0 = w0a0_ref[...]
    w0aL = w0aL_ref[...]
    b0a = b0a_ref[...]
    w0b = w0b_ref[...]
    b0b = b0b_ref[...]

    xWT = jnp.dot(w1a0T_ref[...], xT, preferred_element_type=f32)
    xWTb2 = xWT + b1a + w1a128
    Btbf = Bt.astype(bf16)

    def per_v(j, carry):
        v0 = j * U
        for u in range(U):
            v = v0 + u
            m_row = B_s[pl.ds(v, 1), :]
            g_row = G_s[pl.ds(v, 1), :]
            L = (xWT * m_row).astype(bf16)
            QT = jnp.dot(L, Btbf, preferred_element_type=f32)
            hT = jnp.maximum((xWTb2 + QT) + wsum * g_row, 0.0)
            mh = jax.lax.dot_general(m_row, hT, nt,
                                     preferred_element_type=f32)
            MH_s[pl.ds(v, 1), :] = mh
        return carry

    jax.lax.fori_loop(jnp.int32(0), jnp.int32(N // U), per_v, jnp.int32(0))

    MH = MH_s[...]
    HVU = jnp.dot(MH, w1b0, preferred_element_type=f32) + kdeg * b1b
    HVUL = jnp.dot(MH, w1bL, preferred_element_type=f32) + kdeg * b1bL
    AGG = x_ref[...] + HVU
    H0 = jnp.maximum(
        jnp.dot(AGG, w0a0, preferred_element_type=f32)
        + HVUL * w0aL + b0a, 0.0)
    out_ref[...] = jnp.dot(H0, w0b, preferred_element_type=f32) + b0b


def kernel(x, edge_index, batch, W0a, b0a, W0b, b0b, W1a, b1a, W1b, b1b):
    f32 = jnp.float32
    x = jnp.asarray(x, f32)
    ei = jnp.asarray(edge_index, jnp.int32)
    eip = jnp.zeros((8, E), jnp.int32).at[:2, :].set(ei)

    args = (
        eip, x, x.T,
        jnp.asarray(W1a[:F, :].T, f32),
        jnp.asarray(W1a[F, :][:, None], f32),
        jnp.asarray(W1a[F + 1, :][:, None], f32),
        jnp.asarray(b1a[:, None], f32),
        jnp.asarray(W1b[:, :F], f32),
        jnp.asarray(W1b[:, F][:, None], f32),
        jnp.asarray(b1b[:F][None, :], f32),
        jnp.asarray(b1b[F].reshape(1, 1), f32),
        jnp.asarray(W0a[:F, :], f32),
        jnp.asarray(W0a[F, :][None, :], f32),
        jnp.asarray(b0a[None, :], f32),
        jnp.asarray(W0b, f32),
        jnp.asarray(b0b[None, :], f32),
    )
    return pl.pallas_call(
        _body,
        out_shape=jax.ShapeDtypeStruct((N, F), f32),
        scratch_shapes=[pltpu.VMEM((N, N), f32), pltpu.VMEM((N, N), f32),
                        pltpu.VMEM((N, F), f32)],
    )(*args)


# speedup vs baseline: 1.2591x; 1.2591x over previous
import jax
import jax.numpy as jnp
from jax.experimental import pallas as pl
from jax.experimental.pallas import tpu as pltpu

N = 256
E = 1024
F = 128
U = 32
KA = 264


def _body(eip_ref, x_ref, xT_ref,
          w1a0T_ref, w1a128_ref, w1a129_ref, b1a_ref,
          w1b0_ref, w1bL_ref, b1b_ref, b1bL_ref,
          w0a0_ref, w0aL_ref, b0a_ref, w0b_ref, b0b_ref,
          out_ref, B_s, G_s, MH_s):
    f32 = jnp.float32
    bf16 = jnp.bfloat16
    src_row = eip_ref[0:1, :]
    dst_row = eip_ref[1:2, :]
    xT = xT_ref[...]

    node_iota = jax.lax.broadcasted_iota(jnp.int32, (N, E), 0)
    Sd = (node_iota == dst_row).astype(bf16)
    Ss = (node_iota == src_row).astype(bf16)
    nt = (((1,), (1,)), ((), ()))
    cnt = jax.lax.dot_general(Sd, Ss, nt, preferred_element_type=f32)
    cntT = jax.lax.dot_general(Ss, Sd, nt, preferred_element_type=f32)
    r = jax.lax.broadcasted_iota(jnp.int32, (N, N), 0)
    c = jax.lax.broadcasted_iota(jnp.int32, (N, N), 1)
    offdiag = (r != c).astype(f32)
    B = (cnt > 0).astype(f32) * offdiag
    Bt = (cntT > 0).astype(f32) * offdiag
    Bbf = B.astype(bf16)
    G = jax.lax.dot_general(Bbf, Bbf, nt, preferred_element_type=f32)
    B_s[...] = B
    G_s[...] = G
    kdeg = jnp.sum(B, axis=1, keepdims=True)

    w1a128 = w1a128_ref[...]
    wsum = w1a128 + w1a129_ref[...]
    b1a = b1a_ref[...]
    w1b0 = w1b0_ref[...]
    w1bL = w1bL_ref[...]
    b1b = b1b_ref[...]
    b1bL = b1bL_ref[...]
    w0a0 = w0a0_ref[...]
    w0aL = w0aL_ref[...]
    b0a = b0a_ref[...]
    w0b = w0b_ref[...]
    b0b = b0b_ref[...]

    xWT = jnp.dot(w1a0T_ref[...], xT, preferred_element_type=f32)
    xWTb2 = xWT + b1a + w1a128
    Btbf = Bt.astype(bf16)

    def per_v(j, carry):
        v0 = j * U
        for u in range(U):
            v = v0 + u
            m_row = B_s[pl.ds(v, 1), :]
            g_row = G_s[pl.ds(v, 1), :]
            L = (xWT * m_row).astype(bf16)
            QT = jnp.dot(L, Btbf, preferred_element_type=f32)
            hT = jnp.maximum((xWTb2 + QT) + wsum * g_row, 0.0)
            mh = jax.lax.dot_general(m_row, hT, nt,
                                     preferred_element_type=f32)
            MH_s[pl.ds(v, 1), :] = mh
        return carry

    jax.lax.fori_loop(jnp.int32(0), jnp.int32(N // U), per_v, jnp.int32(0))

    MH = MH_s[...]
    HVU = jnp.dot(MH, w1b0, preferred_element_type=f32) + kdeg * b1b
    HVUL = jnp.dot(MH, w1bL, preferred_element_type=f32) + kdeg * b1bL
    AGG = x_ref[...] + HVU
    H0 = jnp.maximum(
        jnp.dot(AGG, w0a0, preferred_element_type=f32)
        + HVUL * w0aL + b0a, 0.0)
    out_ref[...] = jnp.dot(H0, w0b, preferred_element_type=f32) + b0b


def kernel(x, edge_index, batch, W0a, b0a, W0b, b0b, W1a, b1a, W1b, b1b):
    f32 = jnp.float32
    x = jnp.asarray(x, f32)
    ei = jnp.asarray(edge_index, jnp.int32)
    eip = jnp.zeros((8, E), jnp.int32).at[:2, :].set(ei)

    args = (
        eip, x, x.T,
        jnp.asarray(W1a[:F, :].T, f32),
        jnp.asarray(W1a[F, :][:, None], f32),
        jnp.asarray(W1a[F + 1, :][:, None], f32),
        jnp.asarray(b1a[:, None], f32),
        jnp.asarray(W1b[:, :F], f32),
        jnp.asarray(W1b[:, F][:, None], f32),
        jnp.asarray(b1b[:F][None, :], f32),
        jnp.asarray(b1b[F].reshape(1, 1), f32),
        jnp.asarray(W0a[:F, :], f32),
        jnp.asarray(W0a[F, :][None, :], f32),
        jnp.asarray(b0a[None, :], f32),
        jnp.asarray(W0b, f32),
        jnp.asarray(b0b[None, :], f32),
    )
    return pl.pallas_call(
        _body,
        out_shape=jax.ShapeDtypeStruct((N, F), f32),
        scratch_shapes=[pltpu.VMEM((N, N), f32), pltpu.VMEM((N, N), f32),
                        pltpu.VMEM((N, F), f32)],
    )(*args)


# speedup vs baseline: 1.3052x; 1.0366x over previous
import jax
import jax.numpy as jnp
from jax.experimental import pallas as pl
from jax.experimental.pallas import tpu as pltpu

N = 256
E = 1024
F = 128
U = 64
KA = 264


def _body(eip_ref, x_ref, xT_ref,
          w1a0T_ref, w1a128_ref, w1a129_ref, b1a_ref,
          w1b0_ref, w1bL_ref, b1b_ref, b1bL_ref,
          w0a0_ref, w0aL_ref, b0a_ref, w0b_ref, b0b_ref,
          out_ref, B_s, G_s, MH_s):
    f32 = jnp.float32
    bf16 = jnp.bfloat16
    src_row = eip_ref[0:1, :]
    dst_row = eip_ref[1:2, :]
    xT = xT_ref[...]

    node_iota = jax.lax.broadcasted_iota(jnp.int32, (N, E), 0)
    Sd = (node_iota == dst_row).astype(bf16)
    Ss = (node_iota == src_row).astype(bf16)
    nt = (((1,), (1,)), ((), ()))
    cnt = jax.lax.dot_general(Sd, Ss, nt, preferred_element_type=f32)
    cntT = jax.lax.dot_general(Ss, Sd, nt, preferred_element_type=f32)
    r = jax.lax.broadcasted_iota(jnp.int32, (N, N), 0)
    c = jax.lax.broadcasted_iota(jnp.int32, (N, N), 1)
    offdiag = (r != c).astype(f32)
    B = (cnt > 0).astype(f32) * offdiag
    Bt = (cntT > 0).astype(f32) * offdiag
    Bbf = B.astype(bf16)
    G = jax.lax.dot_general(Bbf, Bbf, nt, preferred_element_type=f32)
    B_s[...] = B
    G_s[...] = G
    kdeg = jnp.sum(B, axis=1, keepdims=True)

    w1a128 = w1a128_ref[...]
    wsum = w1a128 + w1a129_ref[...]
    b1a = b1a_ref[...]
    w1b0 = w1b0_ref[...]
    w1bL = w1bL_ref[...]
    b1b = b1b_ref[...]
    b1bL = b1bL_ref[...]
    w0a0 = w0a0_ref[...]
    w0aL = w0aL_ref[...]
    b0a = b0a_ref[...]
    w0b = w0b_ref[...]
    b0b = b0b_ref[...]

    xWT = jnp.dot(w1a0T_ref[...], xT, preferred_element_type=f32)
    xWTb2 = xWT + b1a + w1a128
    Btbf = Bt.astype(bf16)

    def per_v(j, carry):
        v0 = j * U
        for u in range(U):
            v = v0 + u
            m_row = B_s[pl.ds(v, 1), :]
            g_row = G_s[pl.ds(v, 1), :]
            L = (xWT * m_row).astype(bf16)
            QT = jnp.dot(L, Btbf, preferred_element_type=f32)
            hT = jnp.maximum((xWTb2 + QT) + wsum * g_row, 0.0)
            mh = jax.lax.dot_general(m_row, hT, nt,
                                     preferred_element_type=f32)
            MH_s[pl.ds(v, 1), :] = mh
        return carry

    jax.lax.fori_loop(jnp.int32(0), jnp.int32(N // U), per_v, jnp.int32(0))

    MH = MH_s[...]
    HVU = jnp.dot(MH, w1b0, preferred_element_type=f32) + kdeg * b1b
    HVUL = jnp.dot(MH, w1bL, preferred_element_type=f32) + kdeg * b1bL
    AGG = x_ref[...] + HVU
    H0 = jnp.maximum(
        jnp.dot(AGG, w0a0, preferred_element_type=f32)
        + HVUL * w0aL + b0a, 0.0)
    out_ref[...] = jnp.dot(H0, w0b, preferred_element_type=f32) + b0b


def kernel(x, edge_index, batch, W0a, b0a, W0b, b0b, W1a, b1a, W1b, b1b):
    f32 = jnp.float32
    x = jnp.asarray(x, f32)
    ei = jnp.asarray(edge_index, jnp.int32)
    eip = jnp.zeros((8, E), jnp.int32).at[:2, :].set(ei)

    args = (
        eip, x, x.T,
        jnp.asarray(W1a[:F, :].T, f32),
        jnp.asarray(W1a[F, :][:, None], f32),
        jnp.asarray(W1a[F + 1, :][:, None], f32),
        jnp.asarray(b1a[:, None], f32),
        jnp.asarray(W1b[:, :F], f32),
        jnp.asarray(W1b[:, F][:, None], f32),
        jnp.asarray(b1b[:F][None, :], f32),
        jnp.asarray(b1b[F].reshape(1, 1), f32),
        jnp.asarray(W0a[:F, :], f32),
        jnp.asarray(W0a[F, :][None, :], f32),
        jnp.asarray(b0a[None, :], f32),
        jnp.asarray(W0b, f32),
        jnp.asarray(b0b[None, :], f32),
    )
    return pl.pallas_call(
        _body,
        out_shape=jax.ShapeDtypeStruct((N, F), f32),
        scratch_shapes=[pltpu.VMEM((N, N), f32), pltpu.VMEM((N, N), f32),
                        pltpu.VMEM((N, F), f32)],
    )(*args)


# speedup vs baseline: 1.3072x; 1.0016x over previous
import jax
import jax.numpy as jnp
from jax.experimental import pallas as pl
from jax.experimental.pallas import tpu as pltpu

N = 256
E = 1024
F = 128
U = 64


def _body(eip_ref, x_ref, xT_ref,
          w1a0T_ref, w1a128_ref, w1a129_ref, b1a_ref,
          w1b0_ref, w1bL_ref, b1b_ref, b1bL_ref,
          w0a0_ref, w0aL_ref, b0a_ref, w0b_ref, b0b_ref,
          out_ref, B_s, G_s, MH_s):
    f32 = jnp.float32
    bf16 = jnp.bfloat16
    src_row = eip_ref[0:1, :]
    dst_row = eip_ref[1:2, :]
    xT = xT_ref[...]

    node_iota = jax.lax.broadcasted_iota(jnp.int32, (N, E), 0)
    Sd = (node_iota == dst_row).astype(bf16)
    Ss = (node_iota == src_row).astype(bf16)
    nt = (((1,), (1,)), ((), ()))
    cnt = jax.lax.dot_general(Sd, Ss, nt, preferred_element_type=f32)
    cntT = jax.lax.dot_general(Ss, Sd, nt, preferred_element_type=f32)
    r = jax.lax.broadcasted_iota(jnp.int32, (N, N), 0)
    c = jax.lax.broadcasted_iota(jnp.int32, (N, N), 1)
    offdiag = (r != c).astype(f32)
    B = (cnt > 0).astype(f32) * offdiag
    Bt = (cntT > 0).astype(f32) * offdiag
    Bbf = B.astype(bf16)
    G = jax.lax.dot_general(Bbf, Bbf, nt, preferred_element_type=f32)
    B_s[...] = B
    G_s[...] = G
    kdeg = jnp.sum(B, axis=1, keepdims=True)

    w1a128 = w1a128_ref[...]
    wsum = w1a128 + w1a129_ref[...]
    b1a = b1a_ref[...]
    w1b0 = w1b0_ref[...]
    w1bL = w1bL_ref[...]
    b1b = b1b_ref[...]
    b1bL = b1bL_ref[...]
    w0a0 = w0a0_ref[...]
    w0aL = w0aL_ref[...]
    b0a = b0a_ref[...]
    w0b = w0b_ref[...]
    b0b = b0b_ref[...]

    xWT = jnp.dot(w1a0T_ref[...], xT, preferred_element_type=f32)
    xWTb2 = xWT + b1a + w1a128
    Btbf = Bt.astype(bf16)

    def per_v(j, carry):
        v0 = j * U
        for u in range(U):
            v = v0 + u
            m_row = B_s[pl.ds(v, 1), :]
            g_row = G_s[pl.ds(v, 1), :]
            L = (xWT * m_row).astype(bf16)
            QT = jnp.dot(L, Btbf, preferred_element_type=f32)
            hT = jnp.maximum((xWTb2 + QT) + wsum * g_row, 0.0)
            mh = jax.lax.dot_general(m_row, hT, nt,
                                     preferred_element_type=f32)
            MH_s[pl.ds(v, 1), :] = mh
        return carry

    jax.lax.fori_loop(jnp.int32(0), jnp.int32(N // U), per_v, jnp.int32(0))

    MH = MH_s[...]
    HVU = jnp.dot(MH, w1b0, preferred_element_type=f32) + kdeg * b1b
    HVUL = jnp.dot(MH, w1bL, preferred_element_type=f32) + kdeg * b1bL
    AGG = x_ref[...] + HVU
    H0 = jnp.maximum(
        jnp.dot(AGG, w0a0, preferred_element_type=f32)
        + HVUL * w0aL + b0a, 0.0)
    out_ref[...] = jnp.dot(H0, w0b, preferred_element_type=f32) + b0b


def kernel(x, edge_index, batch, W0a, b0a, W0b, b0b, W1a, b1a, W1b, b1b):
    f32 = jnp.float32
    x = jnp.asarray(x, f32)
    ei = jnp.asarray(edge_index, jnp.int32)
    eip = jnp.zeros((8, E), jnp.int32).at[:2, :].set(ei)

    args = (
        eip, x, x.T,
        jnp.asarray(W1a[:F, :].T, f32),
        jnp.asarray(W1a[F, :][:, None], f32),
        jnp.asarray(W1a[F + 1, :][:, None], f32),
        jnp.asarray(b1a[:, None], f32),
        jnp.asarray(W1b[:, :F], f32),
        jnp.asarray(W1b[:, F][:, None], f32),
        jnp.asarray(b1b[:F][None, :], f32),
        jnp.asarray(b1b[F].reshape(1, 1), f32),
        jnp.asarray(W0a[:F, :], f32),
        jnp.asarray(W0a[F, :][None, :], f32),
        jnp.asarray(b0a[None, :], f32),
        jnp.asarray(W0b, f32),
        jnp.asarray(b0b[None, :], f32),
    )
    return pl.pallas_call(
        _body,
        out_shape=jax.ShapeDtypeStruct((N, F), f32),
        scratch_shapes=[pltpu.VMEM((N, N), f32), pltpu.VMEM((N, N), f32),
                        pltpu.VMEM((N, F), f32)],
    )(*args)
